# baseline (speedup 1.0000x reference)
import jax
import jax.numpy as jnp
from jax.experimental import pallas as pl
from jax.experimental.pallas import tpu as pltpu


def _copy_kernel(x_ref, o_ref):
    o_ref[...] = x_ref[...]


def kernel(x, w1, b1, w2, b2):
    N, C, H, W = x.shape
    HW = H * W
    R = HW * C // 128
    xt = x.transpose(0, 2, 3, 1).reshape(N, R, 128)
    nb = 4
    out = pl.pallas_call(
        _copy_kernel,
        out_shape=jax.ShapeDtypeStruct((N, R, 128), x.dtype),
        grid=(N // nb,),
        in_specs=[pl.BlockSpec((nb, R, 128), lambda n: (n, 0, 0))],
        out_specs=pl.BlockSpec((nb, R, 128), lambda n: (n, 0, 0)),
        compiler_params=pltpu.CompilerParams(
            dimension_semantics=("parallel",),
            vmem_limit_bytes=56 * 1024 * 1024),
    )(xt)
    return out.reshape(N, H, W, C).transpose(0, 3, 1, 2)


# speedup vs baseline: 6.2143x; 6.2143x over previous
import functools

import jax
import jax.numpy as jnp
from jax.experimental import pallas as pl
from jax.experimental.pallas import tpu as pltpu

_VMEM_LIMIT = 60 * 1024 * 1024


def _se_kernel(x_ref, w1t_ref, b1_ref, w2t_ref, b2_ref, o_ref, *, inv_hw):
    xb = x_ref[...]
    pooled = jnp.sum(xb, axis=0) * inv_hw

    h = jnp.dot(pooled, w1t_ref[...],
                preferred_element_type=jnp.float32)
    h = jnp.maximum(h + b1_ref[...], 0.0)
    z = jnp.dot(h, w2t_ref[...],
                preferred_element_type=jnp.float32)
    y = 0.5 * jnp.tanh(0.5 * (z + b2_ref[...])) + 0.5

    o_ref[...] = xb * y[None, :, :]


def kernel(x, w1, b1, w2, b2):
    N, C, H, W = x.shape
    Cr = w1.shape[0]
    HW = H * W

    xv = x.transpose(2, 3, 0, 1).reshape(HW, N, C)
    w1t = w1.T
    w2t = w2.T
    b1r = b1.reshape(1, Cr)
    b2r = b2.reshape(1, C)

    nb = 8
    while N % nb:
        nb //= 2

    out = pl.pallas_call(
        functools.partial(_se_kernel, inv_hw=1.0 / HW),
        out_shape=jax.ShapeDtypeStruct((HW, N, C), x.dtype),
        grid=(N // nb,),
        in_specs=[
            pl.BlockSpec((HW, nb, C), lambda n: (0, n, 0)),
            pl.BlockSpec((C, Cr), lambda n: (0, 0)),
            pl.BlockSpec((1, Cr), lambda n: (0, 0)),
            pl.BlockSpec((Cr, C), lambda n: (0, 0)),
            pl.BlockSpec((1, C), lambda n: (0, 0)),
        ],
        out_specs=pl.BlockSpec((HW, nb, C), lambda n: (0, n, 0)),
        compiler_params=pltpu.CompilerParams(
            dimension_semantics=("parallel",),
            vmem_limit_bytes=_VMEM_LIMIT),
    )(xv, w1t, b1r, w2t, b2r)

    return out.reshape(H, W, N, C).transpose(2, 3, 0, 1)


# speedup vs baseline: 6.2394x; 1.0040x over previous
import functools

import jax
import jax.numpy as jnp
from jax.experimental import pallas as pl
from jax.experimental.pallas import tpu as pltpu

_VMEM_LIMIT = 60 * 1024 * 1024


def _se_kernel(x_ref, w1t_ref, b1_ref, w2t_ref, b2_ref, o_ref, *, inv_hw):
    xb = x_ref[...]
    pooled = jnp.sum(xb, axis=0) * inv_hw

    h = jnp.dot(pooled, w1t_ref[...],
                preferred_element_type=jnp.float32)
    h = jnp.maximum(h + b1_ref[...], 0.0)
    z = jnp.dot(h, w2t_ref[...],
                preferred_element_type=jnp.float32)
    y = 0.5 * jnp.tanh(0.5 * (z + b2_ref[...])) + 0.5

    o_ref[...] = xb * y[None, :, :]


def kernel(x, w1, b1, w2, b2):
    N, C, H, W = x.shape
    Cr = w1.shape[0]
    HW = H * W

    xv = x.transpose(2, 3, 0, 1).reshape(HW, N, C)
    w1t = w1.T
    w2t = w2.T
    b1r = b1.reshape(1, Cr)
    b2r = b2.reshape(1, C)

    nb = 8 if N % 8 == 0 else N

    out = pl.pallas_call(
        functools.partial(_se_kernel, inv_hw=1.0 / HW),
        out_shape=jax.ShapeDtypeStruct((HW, N, C), x.dtype),
        grid=(N // nb,),
        in_specs=[
            pl.BlockSpec((HW, nb, C), lambda n: (0, n, 0)),
            pl.BlockSpec((C, Cr), lambda n: (0, 0)),
            pl.BlockSpec((1, Cr), lambda n: (0, 0)),
            pl.BlockSpec((Cr, C), lambda n: (0, 0)),
            pl.BlockSpec((1, C), lambda n: (0, 0)),
        ],
        out_specs=pl.BlockSpec((HW, nb, C), lambda n: (0, n, 0)),
        compiler_params=pltpu.CompilerParams(
            dimension_semantics=("parallel",),
            vmem_limit_bytes=_VMEM_LIMIT),
    )(xv, w1t, b1r, w2t, b2r)

    return out.reshape(H, W, N, C).transpose(2, 3, 0, 1)
